# Initial kernel scaffold; baseline (speedup 1.0000x reference)
import jax
import jax.numpy as jnp
from jax.experimental import pallas as pl


def kernel(x, edge_index, edge_attr, W_msg, b_msg, W1, b1, W2, b2, Wp, bp):
    raise NotImplementedError("write your pallas kernel here")


# speedup vs baseline: 2.5402x; 2.5402x over previous
import functools

import jax
import jax.numpy as jnp
from jax import lax
from jax.experimental import pallas as pl
from jax.experimental.pallas import tpu as pltpu
from jax.experimental.pallas import tpu_sc as plsc

N_NODES = 10000
N_EDGES = 320000
NPAD = 10112
RPT = 632
JUNK_ROW = 10056
NTILES = 32
EPW = 10240
EPAD = NTILES * EPW
KB_M = 64
NB_M = EPW // KB_M
KB_P = 128
NB_P = EPW // KB_P


@functools.lru_cache(maxsize=None)
def _mesh():
    return plsc.VectorSubcoreMesh(core_axis_name="c", subcore_axis_name="s")


def _dot(a, b):
    return lax.dot_general(a, b, (((1,), (0,)), ((), ())),
                           preferred_element_type=jnp.float32)


def _mm(x, w, b, relu, bm, kc=None):
    m, k = x.shape
    n = w.shape[1]

    def body(x_ref, w_ref, b_ref, o_ref):
        if kc is None:
            y = _dot(x_ref[...], w_ref[...])
        else:
            y = _dot(x_ref[:, 0:kc], w_ref[0:kc, :])
            for k0 in range(kc, k, kc):
                y = y + _dot(x_ref[:, k0:k0 + kc], w_ref[k0:k0 + kc, :])
        y = y + b_ref[...]
        if relu:
            y = jnp.maximum(y, 0.0)
        o_ref[...] = y

    return pl.pallas_call(
        body,
        grid=(m // bm,),
        in_specs=[pl.BlockSpec((bm, k), lambda i: (i, 0)),
                  pl.BlockSpec((k, n), lambda i: (0, 0)),
                  pl.BlockSpec((1, n), lambda i: (0, 0))],
        out_specs=pl.BlockSpec((bm, n), lambda i: (i, 0)),
        out_shape=jax.ShapeDtypeStruct((m, n), jnp.float32),
    )(x, w, b.reshape(1, -1))


def _final_mm(f, w2, b2, wp, bm, kc):
    m = f.shape[0]
    k = f.shape[1]
    ng = m // bm

    def body(x_ref, w_ref, b_ref, wp_ref, o_ref, accum):
        i = pl.program_id(0)
        if kc is None:
            y = _dot(x_ref[...], w_ref[...])
        else:
            y = _dot(x_ref[:, 0:kc], w_ref[0:kc, :])
            for k0 in range(kc, k, kc):
                y = y + _dot(x_ref[:, k0:k0 + kc], w_ref[k0:k0 + kc, :])
        y = jnp.maximum(y + b_ref[...], 0.0)
        part = y[0:8, :]
        for r0 in range(8, bm, 8):
            part = part + y[r0:r0 + 8, :]

        @pl.when(i == 0)
        def _():
            accum[...] = jnp.zeros_like(accum)

        accum[...] = accum[...] + part

        @pl.when(i == ng - 1)
        def _():
            a = accum[...]
            s = a[0:1, :]
            for r0 in range(1, 8):
                s = s + a[r0:r0 + 1, :]
            o_ref[...] = s * (1.0 / N_NODES)

    return pl.pallas_call(
        body,
        grid=(ng,),
        in_specs=[pl.BlockSpec((bm, k), lambda i: (i, 0)),
                  pl.BlockSpec(w2.shape, lambda i: (0, 0)),
                  pl.BlockSpec((1, w2.shape[1]), lambda i: (0, 0)),
                  pl.BlockSpec(wp.shape, lambda i: (0, 0))],
        out_specs=pl.BlockSpec((1, w2.shape[1]), lambda i: (0, 0)),
        out_shape=jax.ShapeDtypeStruct((1, w2.shape[1]), jnp.float32),
        scratch_shapes=[pltpu.VMEM((8, w2.shape[1]), jnp.float32)],
    )(f, w2, b2.reshape(1, -1), wp)


def _msg_body(src3, dst3, xw, ew4, z128, hout, degout,
              acc, sidx, didx, ea, ga, gb, ones,
              isem, esem, gsem, ssem):
    c = lax.axis_index("c")
    s = lax.axis_index("s")
    wid = c * 16 + s
    sl = pl.ds(s * RPT, RPT)

    pltpu.sync_copy(z128, acc.at[sl])

    def ones_init(i, carry):
        for j in range(8):
            ones[i, pl.ds(j * 16, 16)] = jnp.full((16,), 1.0, jnp.float32)
        return carry
    lax.fori_loop(0, KB_M, ones_init, 0)

    plsc.subcore_barrier()

    def issue_idx(b, slot):
        pltpu.async_copy(src3.at[wid, b], sidx.at[slot], isem)
        pltpu.async_copy(dst3.at[wid, b], didx.at[slot], isem)

    def wait_idx():
        pltpu.make_async_copy(src3.at[wid, 0], sidx.at[0], isem).wait()
        pltpu.make_async_copy(dst3.at[wid, 0], didx.at[0], isem).wait()

    def issue_ew(b):
        pltpu.async_copy(ew4.at[wid, b], ea, esem)

    def issue_gather(b, slot, g):
        pltpu.async_copy(xw.at[sidx.at[slot]], g, gsem)

    def wait_data(g):
        pltpu.make_async_copy(ew4.at[wid, 0], ea, esem).wait()
        pltpu.make_async_copy(xw.at[sidx.at[0]], g, gsem).wait()

    def issue_scatter(slot, g):
        pltpu.async_copy(g, acc.at[didx.at[slot]], ssem, add=True)

    def wait_scatter(g):
        pltpu.make_async_copy(g, acc.at[didx.at[0]], ssem).wait()

    def relu_add(e, g):
        def body(i, carry):
            for j in range(8):
                q = pl.ds(j * 16, 16)
                g[i, q] = jnp.maximum(e[i, q] + g[i, q], 0.0)
            return carry
        lax.fori_loop(0, KB_M, body, 0)

    issue_idx(0, 0)
    issue_idx(1, 1)
    wait_idx()
    issue_ew(0)
    issue_gather(0, 0, ga)

    def step(t, b, gx, gy, first, last):
        wait_data(gx)

        if first is None:
            wait_scatter(gy)
        else:
            @pl.when(jnp.logical_not(first))
            def _():
                wait_scatter(gy)

        def feed_next():
            wait_idx()
            issue_gather(b + 1, (b + 1) % 3, gy)
        if last is None:
            feed_next()
        else:
            @pl.when(jnp.logical_not(last))
            def _():
                feed_next()

        @pl.when(b + 2 < NB_M)
        def _():
            issue_idx(b + 2, (b + 2) % 3)

        relu_add(ea, gx)

        def feed_ew():
            issue_ew(b + 1)
        if last is None:
            feed_ew()
        else:
            @pl.when(jnp.logical_not(last))
            def _():
                feed_ew()

        issue_scatter(b % 3, gx)

    def pair(t, carry):
        step(t, 2 * t, ga, gb, t == 0, None)
        step(t, 2 * t + 1, gb, ga, None, t == NB_M // 2 - 1)
        return carry
    lax.fori_loop(0, NB_M // 2, pair, 0)

    wait_scatter(gb)

    plsc.subcore_barrier()
    pltpu.sync_copy(acc.at[sl], hout.at[c, sl])

    pltpu.sync_copy(z128, acc.at[sl])
    plsc.subcore_barrier()

    issue_idx(0, 0)
    issue_idx(1, 1)

    def dstep(b, carry):
        wait_idx()

        @pl.when(b >= 1)
        def _():
            wait_scatter(ones)

        @pl.when(b + 2 < NB_M)
        def _():
            issue_idx(b + 2, (b + 2) % 3)

        issue_scatter(b % 3, ones)
        return carry
    lax.fori_loop(0, NB_M, dstep, 0)

    wait_scatter(ones)
    plsc.subcore_barrier()
    pltpu.sync_copy(acc.at[sl], degout.at[c, sl])


@functools.lru_cache(maxsize=None)
def _make_msg():
    @functools.partial(
        pl.kernel,
        out_type=(jax.ShapeDtypeStruct((2, NPAD, 128), jnp.float32),
                  jax.ShapeDtypeStruct((2, NPAD, 128), jnp.float32)),
        mesh=_mesh(),
        scratch_types=[
            pltpu.VMEM_SHARED((NPAD, 128), jnp.float32),
            pltpu.VMEM((3, KB_M), jnp.int32),
            pltpu.VMEM((3, KB_M), jnp.int32),
            pltpu.VMEM((KB_M, 128), jnp.float32),
            pltpu.VMEM((KB_M, 128), jnp.float32),
            pltpu.VMEM((KB_M, 128), jnp.float32),
            pltpu.VMEM((KB_M, 128), jnp.float32),
            pltpu.SemaphoreType.DMA,
            pltpu.SemaphoreType.DMA,
            pltpu.SemaphoreType.DMA,
            pltpu.SemaphoreType.DMA,
        ],
        name="gnn_msg_sc",
    )
    def msg_kernel(src3, dst3, xw, ew4, z128, hout, degout, *rest):
        _msg_body(src3, dst3, xw, ew4, z128, hout, degout, *rest)

    return msg_kernel


def _prop_body(nch, src3, dst3, us, z128, out,
               acc, sidx, didx, ga, gb, isem, gsem, ssem):
    c = lax.axis_index("c")
    s = lax.axis_index("s")
    wid = c * 16 + s
    sl = pl.ds(s * RPT, RPT)

    def issue_idx(b, slot):
        pltpu.async_copy(src3.at[wid, b], sidx.at[slot], isem)
        pltpu.async_copy(dst3.at[wid, b], didx.at[slot], isem)

    def wait_idx():
        pltpu.make_async_copy(src3.at[wid, 0], sidx.at[0], isem).wait()
        pltpu.make_async_copy(dst3.at[wid, 0], didx.at[0], isem).wait()

    for ch in range(nch):
        u = us[ch]
        pltpu.sync_copy(z128, acc.at[sl])
        plsc.subcore_barrier()

        issue_idx(0, 0)
        issue_idx(1, 1)
        wait_idx()
        pltpu.async_copy(u.at[sidx.at[0]], ga, gsem)

        def step(t, b, gx, gy, first, last, u=u):
            pltpu.make_async_copy(u.at[sidx.at[0]], gx, gsem).wait()

            def wait_prev():
                pltpu.make_async_copy(gy, acc.at[didx.at[0]], ssem).wait()
            if first is None:
                wait_prev()
            else:
                @pl.when(jnp.logical_not(first))
                def _():
                    wait_prev()

            def feed_next():
                wait_idx()
                pltpu.async_copy(u.at[sidx.at[(b + 1) % 3]], gy, gsem)
            if last is None:
                feed_next()
            else:
                @pl.when(jnp.logical_not(last))
                def _():
                    feed_next()

            @pl.when(b + 2 < NB_P)
            def _():
                issue_idx(b + 2, (b + 2) % 3)

            pltpu.async_copy(gx, acc.at[didx.at[b % 3]], ssem, add=True)

        def pair(t, carry):
            step(t, 2 * t, ga, gb, t == 0, None)
            step(t, 2 * t + 1, gb, ga, None, t == NB_P // 2 - 1)
            return carry
        lax.fori_loop(0, NB_P // 2, pair, 0)

        pltpu.make_async_copy(gb, acc.at[didx.at[0]], ssem).wait()
        plsc.subcore_barrier()
        pltpu.sync_copy(acc.at[sl], out.at[c, ch, sl])


@functools.lru_cache(maxsize=None)
def _make_prop(nch):
    scratch = [
        pltpu.VMEM_SHARED((NPAD, 128), jnp.float32),
        pltpu.VMEM((3, KB_P), jnp.int32),
        pltpu.VMEM((3, KB_P), jnp.int32),
        pltpu.VMEM((KB_P, 128), jnp.float32),
        pltpu.VMEM((KB_P, 128), jnp.float32),
        pltpu.SemaphoreType.DMA,
        pltpu.SemaphoreType.DMA,
        pltpu.SemaphoreType.DMA,
    ]

    @functools.partial(
        pl.kernel,
        out_type=jax.ShapeDtypeStruct((2, nch, NPAD, 128), jnp.float32),
        mesh=_mesh(),
        scratch_types=scratch,
        name=f"gnn_prop{nch}_sc",
    )
    def prop(src3, dst3, *args):
        us = args[:nch]
        z128 = args[nch]
        out = args[nch + 1]
        rest = args[nch + 2:]
        _prop_body(nch, src3, dst3, us, z128, out, *rest)

    return prop


def kernel(x, edge_index, edge_attr, W_msg, b_msg, W1, b1, W2, b2, Wp, bp):
    n = N_NODES
    src = edge_index[0].astype(jnp.int32)
    dst = edge_index[1].astype(jnp.int32)
    npad_e = EPAD - N_EDGES
    src_p = jnp.concatenate([src, jnp.zeros((npad_e,), jnp.int32)])
    dst_p = jnp.concatenate([dst, jnp.full((npad_e,), JUNK_ROW, jnp.int32)])
    src3m = src_p.reshape(NTILES, NB_M, KB_M)
    dst3m = dst_p.reshape(NTILES, NB_M, KB_M)
    src3p = src_p.reshape(NTILES, NB_P, KB_P)
    dst3p = dst_p.reshape(NTILES, NB_P, KB_P)

    ea_pad = jnp.concatenate(
        [edge_attr, jnp.zeros((npad_e, edge_attr.shape[1]), jnp.float32)])

    z128 = jnp.zeros((RPT, 128), jnp.float32)
    zb = jnp.zeros((128,), jnp.float32)

    xw = _mm(x, W_msg[:128], zb, relu=False, bm=1000)
    ew = _mm(ea_pad, W_msg[128:], b_msg, relu=False, bm=2048)
    ew4 = ew.reshape(NTILES, NB_M, KB_M, 128)

    hout, degout = _make_msg()(src3m, dst3m, xw, ew4, z128)
    h_neigh = hout[0, :n] + hout[1, :n]
    deg = degout[0, :n, 0] + degout[1, :n, 0]
    norm = jnp.power(jnp.maximum(deg, 1.0), -0.5)[:, None]

    prop2 = _make_prop(2)
    prop1 = _make_prop(1)

    u0a = x * norm
    u0b = h_neigh * norm
    s0 = prop2(src3p, dst3p, u0a, u0b, z128)
    f1a = norm * (s0[0, 0, :n] + s0[1, 0, :n])
    f1b = norm * (s0[0, 1, :n] + s0[1, 1, :n])
    s1 = prop2(src3p, dst3p, norm * f1a, norm * f1b, z128)
    f2a = norm * (s1[0, 0, :n] + s1[1, 0, :n])
    f2b = norm * (s1[0, 1, :n] + s1[1, 1, :n])

    feats1 = jnp.concatenate([x, h_neigh, f1a, f1b, f2a, f2b], axis=1)
    h1 = _mm(feats1, W1, b1, relu=True, bm=1000, kc=256)

    g1 = prop1(src3p, dst3p, h1 * norm, z128)
    p1 = norm * (g1[0, 0, :n] + g1[1, 0, :n])
    g2 = prop1(src3p, dst3p, p1 * norm, z128)
    p2 = norm * (g2[0, 0, :n] + g2[1, 0, :n])

    feats2 = jnp.concatenate([h1, p1, p2], axis=1)
    hg = _final_mm(feats2, W2, b2, Wp, bm=1000, kc=None)
    return hg @ Wp + bp


# speedup vs baseline: 2.7471x; 1.0814x over previous
import functools

import jax
import jax.numpy as jnp
from jax import lax
from jax.experimental import pallas as pl
from jax.experimental.pallas import tpu as pltpu
from jax.experimental.pallas import tpu_sc as plsc

N_NODES = 10000
N_EDGES = 320000
NPAD = 10112
RPT = 632
JUNK_ROW = 10056
NTILES = 32
EPW = 10240
EPAD = NTILES * EPW
KB_M = 64
NB_M = EPW // KB_M
KB_P = 128
NB_P = EPW // KB_P
TB_M = EPAD // KB_M
NB0_M = 236
NB1_M = (TB_M - 16 * NB0_M) // 16
TB_P = EPAD // KB_P
NB0_P = 118
NB1_P = (TB_P - 16 * NB0_P) // 16


@functools.lru_cache(maxsize=None)
def _mesh():
    return plsc.VectorSubcoreMesh(core_axis_name="c", subcore_axis_name="s")


def _dot(a, b):
    return lax.dot_general(a, b, (((1,), (0,)), ((), ())),
                           preferred_element_type=jnp.float32)


def _mm(x, w, b, relu, bm, kc=None):
    m, k = x.shape
    n = w.shape[1]

    def body(x_ref, w_ref, b_ref, o_ref):
        if kc is None:
            y = _dot(x_ref[...], w_ref[...])
        else:
            y = _dot(x_ref[:, 0:kc], w_ref[0:kc, :])
            for k0 in range(kc, k, kc):
                y = y + _dot(x_ref[:, k0:k0 + kc], w_ref[k0:k0 + kc, :])
        y = y + b_ref[...]
        if relu:
            y = jnp.maximum(y, 0.0)
        o_ref[...] = y

    return pl.pallas_call(
        body,
        grid=(m // bm,),
        in_specs=[pl.BlockSpec((bm, k), lambda i: (i, 0)),
                  pl.BlockSpec((k, n), lambda i: (0, 0)),
                  pl.BlockSpec((1, n), lambda i: (0, 0))],
        out_specs=pl.BlockSpec((bm, n), lambda i: (i, 0)),
        out_shape=jax.ShapeDtypeStruct((m, n), jnp.float32),
    )(x, w, b.reshape(1, -1))


def _final_mm(f, w2, b2, wp, bm, kc):
    m = f.shape[0]
    k = f.shape[1]
    ng = m // bm

    def body(x_ref, w_ref, b_ref, wp_ref, o_ref, accum):
        i = pl.program_id(0)
        if kc is None:
            y = _dot(x_ref[...], w_ref[...])
        else:
            y = _dot(x_ref[:, 0:kc], w_ref[0:kc, :])
            for k0 in range(kc, k, kc):
                y = y + _dot(x_ref[:, k0:k0 + kc], w_ref[k0:k0 + kc, :])
        y = jnp.maximum(y + b_ref[...], 0.0)
        part = y[0:8, :]
        for r0 in range(8, bm, 8):
            part = part + y[r0:r0 + 8, :]

        @pl.when(i == 0)
        def _():
            accum[...] = jnp.zeros_like(accum)

        accum[...] = accum[...] + part

        @pl.when(i == ng - 1)
        def _():
            a = accum[...]
            s = a[0:1, :]
            for r0 in range(1, 8):
                s = s + a[r0:r0 + 1, :]
            o_ref[...] = s * (1.0 / N_NODES)

    return pl.pallas_call(
        body,
        grid=(ng,),
        in_specs=[pl.BlockSpec((bm, k), lambda i: (i, 0)),
                  pl.BlockSpec(w2.shape, lambda i: (0, 0)),
                  pl.BlockSpec((1, w2.shape[1]), lambda i: (0, 0)),
                  pl.BlockSpec(wp.shape, lambda i: (0, 0))],
        out_specs=pl.BlockSpec((1, w2.shape[1]), lambda i: (0, 0)),
        out_shape=jax.ShapeDtypeStruct((1, w2.shape[1]), jnp.float32),
        scratch_shapes=[pltpu.VMEM((8, w2.shape[1]), jnp.float32)],
    )(f, w2, b2.reshape(1, -1), wp)


def _msg_body(src3, dst3, xw, ew4, z128, hout, degout,
              acc, sidx, didx, ea, ga, gb, ones,
              isem, esem, gsem, ssem):
    c = lax.axis_index("c")
    s = lax.axis_index("s")
    base = jnp.where(c == 0, s * NB0_M, 16 * NB0_M + s * NB1_M)
    nb = jnp.where(c == 0, NB0_M, NB1_M)
    sl = pl.ds(s * RPT, RPT)

    pltpu.sync_copy(z128, acc.at[sl])

    def ones_init(i, carry):
        for j in range(8):
            ones[i, pl.ds(j * 16, 16)] = jnp.full((16,), 1.0, jnp.float32)
        return carry
    lax.fori_loop(0, KB_M, ones_init, 0)

    plsc.subcore_barrier()

    def issue_idx(b, slot):
        pltpu.async_copy(src3.at[base + b], sidx.at[slot], isem)
        pltpu.async_copy(dst3.at[base + b], didx.at[slot], isem)

    def wait_idx():
        pltpu.make_async_copy(src3.at[0], sidx.at[0], isem).wait()
        pltpu.make_async_copy(dst3.at[0], didx.at[0], isem).wait()

    def issue_ew(b):
        pltpu.async_copy(ew4.at[base + b], ea, esem)

    def issue_gather(b, slot, g):
        pltpu.async_copy(xw.at[sidx.at[slot]], g, gsem)

    def wait_data(g):
        pltpu.make_async_copy(ew4.at[0], ea, esem).wait()
        pltpu.make_async_copy(xw.at[sidx.at[0]], g, gsem).wait()

    def issue_scatter(slot, g):
        pltpu.async_copy(g, acc.at[didx.at[slot]], ssem, add=True)

    def wait_scatter(g):
        pltpu.make_async_copy(g, acc.at[didx.at[0]], ssem).wait()

    def relu_add(e, g):
        def body(i, carry):
            for j in range(8):
                q = pl.ds(j * 16, 16)
                g[i, q] = jnp.maximum(e[i, q] + g[i, q], 0.0)
            return carry
        lax.fori_loop(0, KB_M, body, 0)

    issue_idx(0, 0)
    issue_idx(1, 1)
    wait_idx()
    issue_ew(0)
    issue_gather(0, 0, ga)

    def step(t, b, gx, gy, first, last):
        wait_data(gx)

        if first is None:
            wait_scatter(gy)
        else:
            @pl.when(jnp.logical_not(first))
            def _():
                wait_scatter(gy)

        def feed_next():
            wait_idx()
            issue_gather(b + 1, (b + 1) % 3, gy)
        if last is None:
            feed_next()
        else:
            @pl.when(jnp.logical_not(last))
            def _():
                feed_next()

        @pl.when(b + 2 < nb)
        def _():
            issue_idx(b + 2, (b + 2) % 3)

        relu_add(ea, gx)

        def feed_ew():
            issue_ew(b + 1)
        if last is None:
            feed_ew()
        else:
            @pl.when(jnp.logical_not(last))
            def _():
                feed_ew()

        issue_scatter(b % 3, gx)

    def pair(t, carry):
        step(t, 2 * t, ga, gb, t == 0, None)
        step(t, 2 * t + 1, gb, ga, None, t == nb // 2 - 1)
        return carry
    lax.fori_loop(0, nb // 2, pair, 0)

    wait_scatter(gb)

    plsc.subcore_barrier()
    pltpu.sync_copy(acc.at[sl], hout.at[c, sl])

    pltpu.sync_copy(z128, acc.at[sl])
    plsc.subcore_barrier()

    issue_idx(0, 0)
    issue_idx(1, 1)

    def dstep(b, carry):
        wait_idx()

        @pl.when(b >= 1)
        def _():
            wait_scatter(ones)

        @pl.when(b + 2 < nb)
        def _():
            issue_idx(b + 2, (b + 2) % 3)

        issue_scatter(b % 3, ones)
        return carry
    lax.fori_loop(0, nb, dstep, 0)

    wait_scatter(ones)
    plsc.subcore_barrier()
    pltpu.sync_copy(acc.at[sl], degout.at[c, sl])


@functools.lru_cache(maxsize=None)
def _make_msg():
    @functools.partial(
        pl.kernel,
        out_type=(jax.ShapeDtypeStruct((2, NPAD, 128), jnp.float32),
                  jax.ShapeDtypeStruct((2, NPAD, 128), jnp.float32)),
        mesh=_mesh(),
        scratch_types=[
            pltpu.VMEM_SHARED((NPAD, 128), jnp.float32),
            pltpu.VMEM((3, KB_M), jnp.int32),
            pltpu.VMEM((3, KB_M), jnp.int32),
            pltpu.VMEM((KB_M, 128), jnp.float32),
            pltpu.VMEM((KB_M, 128), jnp.float32),
            pltpu.VMEM((KB_M, 128), jnp.float32),
            pltpu.VMEM((KB_M, 128), jnp.float32),
            pltpu.SemaphoreType.DMA,
            pltpu.SemaphoreType.DMA,
            pltpu.SemaphoreType.DMA,
            pltpu.SemaphoreType.DMA,
        ],
        name="gnn_msg_sc",
    )
    def msg_kernel(src3, dst3, xw, ew4, z128, hout, degout, *rest):
        _msg_body(src3, dst3, xw, ew4, z128, hout, degout, *rest)

    return msg_kernel


def _prop_body(nch, src3, dst3, us, z128, out,
               acc, sidx, didx, ga, gb, isem, gsem, ssem):
    c = lax.axis_index("c")
    s = lax.axis_index("s")
    base = jnp.where(c == 0, s * NB0_P, 16 * NB0_P + s * NB1_P)
    nb = jnp.where(c == 0, NB0_P, NB1_P)
    sl = pl.ds(s * RPT, RPT)

    def issue_idx(b, slot):
        pltpu.async_copy(src3.at[base + b], sidx.at[slot], isem)
        pltpu.async_copy(dst3.at[base + b], didx.at[slot], isem)

    def wait_idx():
        pltpu.make_async_copy(src3.at[0], sidx.at[0], isem).wait()
        pltpu.make_async_copy(dst3.at[0], didx.at[0], isem).wait()

    for ch in range(nch):
        u = us[ch]
        pltpu.sync_copy(z128, acc.at[sl])
        plsc.subcore_barrier()

        issue_idx(0, 0)
        issue_idx(1, 1)
        wait_idx()
        pltpu.async_copy(u.at[sidx.at[0]], ga, gsem)

        def step(t, b, gx, gy, first, last, u=u):
            pltpu.make_async_copy(u.at[sidx.at[0]], gx, gsem).wait()

            def wait_prev():
                pltpu.make_async_copy(gy, acc.at[didx.at[0]], ssem).wait()
            if first is None:
                wait_prev()
            else:
                @pl.when(jnp.logical_not(first))
                def _():
                    wait_prev()

            def feed_next():
                wait_idx()
                pltpu.async_copy(u.at[sidx.at[(b + 1) % 3]], gy, gsem)
            if last is None:
                feed_next()
            else:
                @pl.when(jnp.logical_not(last))
                def _():
                    feed_next()

            @pl.when(b + 2 < nb)
            def _():
                issue_idx(b + 2, (b + 2) % 3)

            pltpu.async_copy(gx, acc.at[didx.at[b % 3]], ssem, add=True)

        def pair(t, carry):
            step(t, 2 * t, ga, gb, t == 0, None)
            step(t, 2 * t + 1, gb, ga, None, t == nb // 2 - 1)
            return carry
        lax.fori_loop(0, nb // 2, pair, 0)

        pltpu.make_async_copy(gb, acc.at[didx.at[0]], ssem).wait()
        plsc.subcore_barrier()
        pltpu.sync_copy(acc.at[sl], out.at[c, ch, sl])


@functools.lru_cache(maxsize=None)
def _make_prop(nch):
    scratch = [
        pltpu.VMEM_SHARED((NPAD, 128), jnp.float32),
        pltpu.VMEM((3, KB_P), jnp.int32),
        pltpu.VMEM((3, KB_P), jnp.int32),
        pltpu.VMEM((KB_P, 128), jnp.float32),
        pltpu.VMEM((KB_P, 128), jnp.float32),
        pltpu.SemaphoreType.DMA,
        pltpu.SemaphoreType.DMA,
        pltpu.SemaphoreType.DMA,
    ]

    @functools.partial(
        pl.kernel,
        out_type=jax.ShapeDtypeStruct((2, nch, NPAD, 128), jnp.float32),
        mesh=_mesh(),
        scratch_types=scratch,
        name=f"gnn_prop{nch}_sc",
    )
    def prop(src3, dst3, *args):
        us = args[:nch]
        z128 = args[nch]
        out = args[nch + 1]
        rest = args[nch + 2:]
        _prop_body(nch, src3, dst3, us, z128, out, *rest)

    return prop


def kernel(x, edge_index, edge_attr, W_msg, b_msg, W1, b1, W2, b2, Wp, bp):
    n = N_NODES
    src = edge_index[0].astype(jnp.int32)
    dst = edge_index[1].astype(jnp.int32)
    npad_e = EPAD - N_EDGES
    src_p = jnp.concatenate([src, jnp.zeros((npad_e,), jnp.int32)])
    dst_p = jnp.concatenate([dst, jnp.full((npad_e,), JUNK_ROW, jnp.int32)])
    src3m = src_p.reshape(TB_M, KB_M)
    dst3m = dst_p.reshape(TB_M, KB_M)
    src3p = src_p.reshape(TB_P, KB_P)
    dst3p = dst_p.reshape(TB_P, KB_P)

    ea_pad = jnp.concatenate(
        [edge_attr, jnp.zeros((npad_e, edge_attr.shape[1]), jnp.float32)])

    z128 = jnp.zeros((RPT, 128), jnp.float32)
    zb = jnp.zeros((128,), jnp.float32)

    xw = _mm(x, W_msg[:128], zb, relu=False, bm=1000)
    ew = _mm(ea_pad, W_msg[128:], b_msg, relu=False, bm=2048)
    ew4 = ew.reshape(TB_M, KB_M, 128)

    hout, degout = _make_msg()(src3m, dst3m, xw, ew4, z128)
    h_neigh = hout[0, :n] + hout[1, :n]
    deg = degout[0, :n, 0] + degout[1, :n, 0]
    norm = jnp.power(jnp.maximum(deg, 1.0), -0.5)[:, None]

    prop2 = _make_prop(2)
    prop1 = _make_prop(1)

    u0a = x * norm
    u0b = h_neigh * norm
    s0 = prop2(src3p, dst3p, u0a, u0b, z128)
    f1a = norm * (s0[0, 0, :n] + s0[1, 0, :n])
    f1b = norm * (s0[0, 1, :n] + s0[1, 1, :n])
    s1 = prop2(src3p, dst3p, norm * f1a, norm * f1b, z128)
    f2a = norm * (s1[0, 0, :n] + s1[1, 0, :n])
    f2b = norm * (s1[0, 1, :n] + s1[1, 1, :n])

    feats1 = jnp.concatenate([x, h_neigh, f1a, f1b, f2a, f2b], axis=1)
    h1 = _mm(feats1, W1, b1, relu=True, bm=1000, kc=256)

    g1 = prop1(src3p, dst3p, h1 * norm, z128)
    p1 = norm * (g1[0, 0, :n] + g1[1, 0, :n])
    g2 = prop1(src3p, dst3p, p1 * norm, z128)
    p2 = norm * (g2[0, 0, :n] + g2[1, 0, :n])

    feats2 = jnp.concatenate([h1, p1, p2], axis=1)
    hg = _final_mm(feats2, W2, b2, Wp, bm=1000, kc=None)
    return hg @ Wp + bp
